# baseline (speedup 1.0000x reference)
import functools

import jax
import jax.numpy as jnp
from jax import lax
from jax.experimental import pallas as pl
from jax.experimental.pallas import tpu as pltpu
from jax.experimental.pallas import tpu_sc as plsc

_B = 16384
_D = 64
_NC = 2
_NS = 16
_NW = _NC * _NS
_BPW = _B // _NW
_IDXW = 128
_NCHUNK = _BPW // _IDXW

_TB = 1024


def _build_gather():
    mesh = plsc.VectorSubcoreMesh(core_axis_name="c", subcore_axis_name="s")

    @functools.partial(
        pl.kernel,
        mesh=mesh,
        out_type=(
            jax.ShapeDtypeStruct((_B, _D), jnp.float32),
            jax.ShapeDtypeStruct((_B, _D), jnp.float32),
        ),
        scratch_types=[
            pltpu.VMEM((_NCHUNK, _IDXW), jnp.int32),
            pltpu.VMEM((_NCHUNK, _IDXW), jnp.int32),
            pltpu.VMEM((_BPW, _D), jnp.float32),
            pltpu.VMEM((_BPW, _D), jnp.float32),
            pltpu.SemaphoreType.DMA,
        ],
        compiler_params=pltpu.CompilerParams(use_tc_tiling_on_sc=False),
    )
    def gather(user_hbm, movie_hbm, ut_hbm, mt_hbm, ue_hbm, me_hbm,
               uidx, midx, urows, mrows, sem):
        wid = lax.axis_index("s") * _NC + lax.axis_index("c")
        base = wid * _BPW
        pltpu.sync_copy(user_hbm.at[pl.ds(wid * _NCHUNK, _NCHUNK)], uidx)
        pltpu.sync_copy(movie_hbm.at[pl.ds(wid * _NCHUNK, _NCHUNK)], midx)
        copies = []
        for j in range(_NCHUNK):
            copies.append(pltpu.async_copy(
                ut_hbm.at[uidx.at[j]], urows.at[pl.ds(j * _IDXW, _IDXW)], sem))
            copies.append(pltpu.async_copy(
                mt_hbm.at[midx.at[j]], mrows.at[pl.ds(j * _IDXW, _IDXW)], sem))
        for c in copies:
            c.wait()
        pltpu.sync_copy(urows, ue_hbm.at[pl.ds(base, _BPW)])
        pltpu.sync_copy(mrows, me_hbm.at[pl.ds(base, _BPW)])

    return gather


def _mlp_body(ue, me, w1u, w1m, b1, w2, b2, w3, b3, w4, b4, w5, b5, out):
    f32 = jnp.float32
    x = jnp.dot(ue[...], w1u[...], preferred_element_type=f32)
    x = x + jnp.dot(me[...], w1m[...], preferred_element_type=f32)
    x = jnp.maximum(x + b1[...], 0.0)
    x = jnp.maximum(jnp.dot(x, w2[...], preferred_element_type=f32) + b2[...], 0.0)
    x = jnp.maximum(jnp.dot(x, w3[...], preferred_element_type=f32) + b3[...], 0.0)
    x = jnp.maximum(jnp.dot(x, w4[...], preferred_element_type=f32) + b4[...], 0.0)
    out[...] = jnp.dot(x, w5[...], preferred_element_type=f32) + b5[...]


def _full(shape):
    return pl.BlockSpec(shape, lambda i: (0, 0))


def _mlp(ue, me, w1u, w1m, b1, w2, b2, w3, b3, w4, b4, w5, b5):
    nblk = _B // _TB
    return pl.pallas_call(
        _mlp_body,
        grid=(nblk,),
        in_specs=[
            pl.BlockSpec((_TB, _D), lambda i: (i, 0)),
            pl.BlockSpec((_TB, _D), lambda i: (i, 0)),
            _full(w1u.shape), _full(w1m.shape), _full(b1.shape),
            _full(w2.shape), _full(b2.shape),
            _full(w3.shape), _full(b3.shape),
            _full(w4.shape), _full(b4.shape),
            _full(w5.shape), _full(b5.shape),
        ],
        out_specs=pl.BlockSpec((_TB, 1), lambda i: (i, 0)),
        out_shape=jax.ShapeDtypeStruct((_B, 1), jnp.float32),
    )(ue, me, w1u, w1m, b1, w2, b2, w3, b3, w4, b4, w5, b5)


def kernel(user, movie, user_table, movie_table,
           W1, b1, W2, b2, W3, b3, W4, b4, W5, b5):
    user2d = user.astype(jnp.int32).reshape(_B // _IDXW, _IDXW)
    movie2d = movie.astype(jnp.int32).reshape(_B // _IDXW, _IDXW)
    ue, me = _build_gather()(user2d, movie2d, user_table, movie_table)
    return _mlp(
        ue, me,
        W1[:_D], W1[_D:], b1.reshape(1, -1),
        W2, b2.reshape(1, -1),
        W3, b3.reshape(1, -1),
        W4, b4.reshape(1, -1),
        W5, b5.reshape(1, -1),
    )
